# baseline (device time: 12176 ns/iter reference)
import jax
import jax.numpy as jnp
from jax import lax
from jax.experimental import pallas as pl
from jax.experimental.pallas import tpu as pltpu

C = 8


def kernel(x):
    m, n = x.shape
    h = m // 2
    ck = h // C

    def body(x_hbm, out_ref, xv_ref, in_sems, y_send, y_recv, f_send,
             f_recv):
        my_x = lax.axis_index("x")
        my_y = lax.axis_index("y")
        my_z = lax.axis_index("z")
        s = (my_x + my_z) % 2
        ynbr = (my_x, 1 - my_y, my_z)
        xnbr = (1 - my_x, my_y, my_z)

        dma_s = pltpu.make_async_copy(
            x_hbm.at[pl.ds(s * h, h)], xv_ref.at[pl.ds(s * h, h)],
            in_sems.at[0],
        )
        dma_s.start()
        dma_o = pltpu.make_async_copy(
            x_hbm.at[pl.ds((1 - s) * h, h)],
            xv_ref.at[pl.ds((1 - s) * h, h)],
            in_sems.at[1],
        )
        dma_o.start()

        barrier = pltpu.get_barrier_semaphore()
        for nbr in (ynbr, xnbr):
            pl.semaphore_signal(
                barrier, inc=1, device_id=nbr,
                device_id_type=pl.DeviceIdType.MESH,
            )
        pl.semaphore_wait(barrier, 2)

        my_off = my_y * m
        my_half = my_off + s * h
        dma_s.wait()
        out_ref[pl.ds(my_half, h), :] = (
            xv_ref[pl.ds(s * h, h), :].astype(jnp.bfloat16)
        )

        y_rdmas = []
        for c in range(C):
            off = my_half + c * ck
            r = pltpu.make_async_remote_copy(
                src_ref=out_ref.at[pl.ds(off, ck)],
                dst_ref=out_ref.at[pl.ds(off, ck)],
                send_sem=y_send.at[c],
                recv_sem=y_recv.at[c],
                device_id=ynbr,
                device_id_type=pl.DeviceIdType.MESH,
            )
            r.start()
            y_rdmas.append(r)

        dma_o.wait()
        out_ref[pl.ds(my_off + (1 - s) * h, h), :] = (
            xv_ref[pl.ds((1 - s) * h, h), :].astype(jnp.bfloat16)
        )

        rem_half = (1 - my_y) * m + s * h
        f_rdmas = []
        for c in range(C):
            off = rem_half + c * ck
            recv = pltpu.make_async_remote_copy(
                src_ref=out_ref.at[pl.ds(off, ck)],
                dst_ref=out_ref.at[pl.ds(off, ck)],
                send_sem=y_send.at[c],
                recv_sem=y_recv.at[c],
                device_id=ynbr,
                device_id_type=pl.DeviceIdType.MESH,
            )
            recv.wait_recv()
            f = pltpu.make_async_remote_copy(
                src_ref=out_ref.at[pl.ds(off, ck)],
                dst_ref=out_ref.at[pl.ds(off, ck)],
                send_sem=f_send.at[c],
                recv_sem=f_recv.at[c],
                device_id=xnbr,
                device_id_type=pl.DeviceIdType.MESH,
            )
            f.start()
            f_rdmas.append(f)

        fwd_in = (1 - my_y) * m + (1 - s) * h
        for c in range(C):
            off = fwd_in + c * ck
            rin = pltpu.make_async_remote_copy(
                src_ref=out_ref.at[pl.ds(off, ck)],
                dst_ref=out_ref.at[pl.ds(off, ck)],
                send_sem=f_send.at[c],
                recv_sem=f_recv.at[c],
                device_id=xnbr,
                device_id_type=pl.DeviceIdType.MESH,
            )
            rin.wait_recv()

        for r in y_rdmas:
            r.wait_send()
        for r in f_rdmas:
            r.wait_send()

    return pl.pallas_call(
        body,
        out_shape=jax.ShapeDtypeStruct((2 * m, n), jnp.bfloat16),
        in_specs=[pl.BlockSpec(memory_space=pl.ANY)],
        out_specs=pl.BlockSpec(memory_space=pltpu.VMEM),
        scratch_shapes=[
            pltpu.VMEM((m, n), jnp.float32),
            pltpu.SemaphoreType.DMA((2,)),
            pltpu.SemaphoreType.DMA((C,)),
            pltpu.SemaphoreType.DMA((C,)),
            pltpu.SemaphoreType.DMA((C,)),
            pltpu.SemaphoreType.DMA((C,)),
        ],
        compiler_params=pltpu.CompilerParams(collective_id=0),
    )(x)


# device time: 10643 ns/iter; 1.1440x vs baseline; 1.1440x over previous
import jax
import jax.numpy as jnp
from jax import lax
from jax.experimental import pallas as pl
from jax.experimental.pallas import tpu as pltpu

C = 5
CK = 32
FWD = C * CK
OVL = 512 - 2 * FWD


def kernel(x):
    m, n = x.shape
    assert m == 2 * FWD + OVL

    def body(x_ref, out_ref, vstage, vrecv, yf_send, yf_recv, ov_sems,
             f_send, f_recv, ldma_sems):
        my_x = lax.axis_index("x")
        my_y = lax.axis_index("y")
        my_z = lax.axis_index("z")
        s = (my_x + my_z) % 2
        ynbr = (my_x, 1 - my_y, my_z)
        xnbr = (1 - my_x, my_y, my_z)

        fwd_off = s * (FWD + OVL)
        rfwd_off = (1 - s) * (FWD + OVL)
        my_base = my_y * m
        rem_base = (1 - my_y) * m

        barrier = pltpu.get_barrier_semaphore()
        for nbr in (ynbr, xnbr):
            pl.semaphore_signal(
                barrier, inc=1, device_id=nbr,
                device_id_type=pl.DeviceIdType.MESH,
            )
        pl.semaphore_wait(barrier, 2)

        vstage[pl.ds(fwd_off, FWD), :] = (
            x_ref[pl.ds(fwd_off, FWD), :].astype(jnp.bfloat16)
        )
        y_rdmas = []
        for c in range(C):
            r = pltpu.make_async_remote_copy(
                src_ref=vstage.at[pl.ds(fwd_off + c * CK, CK)],
                dst_ref=vrecv.at[pl.ds(c * CK, CK)],
                send_sem=yf_send.at[c],
                recv_sem=yf_recv.at[c],
                device_id=ynbr,
                device_id_type=pl.DeviceIdType.MESH,
            )
            r.start()
            y_rdmas.append(r)

        vstage[pl.ds(FWD, OVL), :] = (
            x_ref[pl.ds(FWD, OVL), :].astype(jnp.bfloat16)
        )
        ov = pltpu.make_async_remote_copy(
            src_ref=vstage.at[pl.ds(FWD, OVL)],
            dst_ref=out_ref.at[pl.ds(my_base + FWD, OVL)],
            send_sem=ov_sems.at[0],
            recv_sem=ov_sems.at[1],
            device_id=ynbr,
            device_id_type=pl.DeviceIdType.MESH,
        )
        ov.start()

        vstage[pl.ds(rfwd_off, FWD), :] = (
            x_ref[pl.ds(rfwd_off, FWD), :].astype(jnp.bfloat16)
        )
        own = pltpu.make_async_copy(
            vstage, out_ref.at[pl.ds(my_base, m)], ldma_sems.at[0]
        )
        own.start()

        f_rdmas = []
        for c in range(C):
            recv = pltpu.make_async_remote_copy(
                src_ref=vstage.at[pl.ds(fwd_off + c * CK, CK)],
                dst_ref=vrecv.at[pl.ds(c * CK, CK)],
                send_sem=yf_send.at[c],
                recv_sem=yf_recv.at[c],
                device_id=ynbr,
                device_id_type=pl.DeviceIdType.MESH,
            )
            recv.wait_recv()
            f = pltpu.make_async_remote_copy(
                src_ref=vrecv.at[pl.ds(c * CK, CK)],
                dst_ref=out_ref.at[pl.ds(rem_base + fwd_off + c * CK, CK)],
                send_sem=f_send.at[c],
                recv_sem=f_recv.at[c],
                device_id=xnbr,
                device_id_type=pl.DeviceIdType.MESH,
            )
            f.start()
            f_rdmas.append(f)

        stg = pltpu.make_async_copy(
            vrecv, out_ref.at[pl.ds(rem_base + fwd_off, FWD)],
            ldma_sems.at[1],
        )
        stg.start()

        ov_in = pltpu.make_async_remote_copy(
            src_ref=vstage.at[pl.ds(FWD, OVL)],
            dst_ref=out_ref.at[pl.ds(rem_base + FWD, OVL)],
            send_sem=ov_sems.at[0],
            recv_sem=ov_sems.at[1],
            device_id=ynbr,
            device_id_type=pl.DeviceIdType.MESH,
        )
        ov_in.wait_recv()

        for c in range(C):
            rin = pltpu.make_async_remote_copy(
                src_ref=vrecv.at[pl.ds(c * CK, CK)],
                dst_ref=out_ref.at[pl.ds(rem_base + rfwd_off + c * CK, CK)],
                send_sem=f_send.at[c],
                recv_sem=f_recv.at[c],
                device_id=xnbr,
                device_id_type=pl.DeviceIdType.MESH,
            )
            rin.wait_recv()

        for r in y_rdmas:
            r.wait_send()
        ov.wait_send()
        for r in f_rdmas:
            r.wait_send()
        own.wait()
        stg.wait()

    return pl.pallas_call(
        body,
        out_shape=jax.ShapeDtypeStruct((2 * m, n), jnp.bfloat16),
        in_specs=[pl.BlockSpec(memory_space=pltpu.VMEM)],
        out_specs=pl.BlockSpec(memory_space=pl.ANY),
        scratch_shapes=[
            pltpu.VMEM((m, n), jnp.bfloat16),
            pltpu.VMEM((FWD, n), jnp.bfloat16),
            pltpu.SemaphoreType.DMA((C,)),
            pltpu.SemaphoreType.DMA((C,)),
            pltpu.SemaphoreType.DMA((2,)),
            pltpu.SemaphoreType.DMA((C,)),
            pltpu.SemaphoreType.DMA((C,)),
            pltpu.SemaphoreType.DMA((2,)),
        ],
        compiler_params=pltpu.CompilerParams(collective_id=0),
    )(x)
